# Initial kernel scaffold; baseline (speedup 1.0000x reference)
import jax
import jax.numpy as jnp
from jax.experimental import pallas as pl


def kernel(x, edge_index, edge_weight, feat_rand1, edge_rand1, feat_rand2, edge_rand2, W1, b1, W2, b2):
    raise NotImplementedError("write your pallas kernel here")


# speedup vs baseline: 7.0742x; 7.0742x over previous
import functools

import jax
import jax.numpy as jnp
from jax import lax
from jax.experimental import pallas as pl
from jax.experimental.pallas import tpu as pltpu
from jax.experimental.pallas import tpu_sc as plsc

_N = 10000
_NP = 10240
_E = 320000
_D = 128

_NC = 2
_NS = 16
_NW = _NC * _NS

_EPW = _E // _NW
_CK = 128
_NCH = _E // _CK
_TRIPS = (_NCH + _NW - 1) // _NW
_STRIPE = _NP // _NS

_RB = 512
_NRB = _NP // _RB

_mesh = plsc.VectorSubcoreMesh(core_axis_name="c", subcore_axis_name="s")


@functools.partial(
    pl.kernel,
    out_type=jax.ShapeDtypeStruct((_NW * 3 * _NP,), jnp.float32),
    mesh=_mesh,
    compiler_params=pltpu.CompilerParams(needs_layout_passes=False),
    scratch_types=[
        pltpu.VMEM((_EPW,), jnp.int32),
        pltpu.VMEM((_EPW,), jnp.float32),
        pltpu.VMEM((_EPW,), jnp.float32),
        pltpu.VMEM((_EPW,), jnp.float32),
        pltpu.VMEM((3 * _NP,), jnp.float32),
    ],
)
def _deg_kernel(dst_h, w_h, em1_h, em2_h, out_h, dstv, wv, e1v, e2v, acc):
    c = lax.axis_index("c")
    s = lax.axis_index("s")
    wid = s * _NC + c
    base = wid * _EPW

    zero = jnp.zeros((16,), jnp.float32)

    def zbody(m, carry):
        acc[pl.ds(m * 16, 16)] = zero
        return carry

    lax.fori_loop(0, (3 * _NP) // 16, zbody, 0)

    pltpu.sync_copy(dst_h.at[pl.ds(base, _EPW)], dstv)
    pltpu.sync_copy(w_h.at[pl.ds(base, _EPW)], wv)
    pltpu.sync_copy(em1_h.at[pl.ds(base, _EPW)], e1v)
    pltpu.sync_copy(em2_h.at[pl.ds(base, _EPW)], e2v)

    def ebody(g, carry):
        o = g * 16
        d16 = dstv[pl.ds(o, 16)]
        w16 = wv[pl.ds(o, 16)]
        e1 = e1v[pl.ds(o, 16)]
        e2 = e2v[pl.ds(o, 16)]
        plsc.addupdate_scatter(acc, [d16], w16)
        plsc.addupdate_scatter(acc, [d16 + _NP], w16, mask=e1 > 0.3)
        plsc.addupdate_scatter(acc, [d16 + 2 * _NP], w16, mask=e2 > 0.3)
        return carry

    lax.fori_loop(0, _EPW // 16, ebody, 0)

    pltpu.sync_copy(acc, out_h.at[pl.ds(wid * (3 * _NP), 3 * _NP)])


@functools.partial(
    pl.kernel,
    out_type=jax.ShapeDtypeStruct((_NC * 3 * _NP, _D), jnp.float32),
    mesh=_mesh,
    compiler_params=pltpu.CompilerParams(needs_layout_passes=False),
    scratch_types=[
        pltpu.VMEM((_CK,), jnp.int32),
        pltpu.VMEM((_CK,), jnp.int32),
        pltpu.VMEM((_CK,), jnp.int32),
        pltpu.VMEM((_CK,), jnp.float32),
        pltpu.VMEM((_CK,), jnp.float32),
        pltpu.VMEM((_CK,), jnp.float32),
        pltpu.VMEM((_CK, _D), jnp.float32),
        pltpu.VMEM((_STRIPE // 5, _D), jnp.float32),
        pltpu.VMEM_SHARED((_NP, _D), jnp.float32),
        pltpu.SemaphoreType.DMA,
    ],
)
def _agg_kernel(table, src_h, dst_h, w_h, em1_h, em2_h, out_h,
                sidx, didx, gidx, wvec, emvec, wsc, rows, zbuf, acc, gsem):
    c = lax.axis_index("c")
    s = lax.axis_index("s")
    wid = s * _NC + c
    row0 = s * _STRIPE

    zero = jnp.zeros((16,), jnp.float32)

    def zb(i, carry):
        for j in range(_D // 16):
            zbuf[i, pl.ds(j * 16, 16)] = zero
        return carry

    lax.fori_loop(0, _STRIPE // 5, zb, 0)

    for v in range(3):
        em_h = (None, em1_h, em2_h)[v]
        for j in range(5):
            pltpu.sync_copy(zbuf, acc.at[pl.ds(row0 + j * (_STRIPE // 5),
                                               _STRIPE // 5)])
        plsc.subcore_barrier()

        def chunk_body(i, carry):
            cid = wid + i * _NW

            @pl.when(cid < _NCH)
            def _():
                base = cid * _CK
                pltpu.sync_copy(src_h.at[pl.ds(base, _CK)], sidx)
                pltpu.sync_copy(dst_h.at[pl.ds(base, _CK)], didx)
                pltpu.sync_copy(w_h.at[pl.ds(base, _CK)], wvec)
                if v > 0:
                    pltpu.sync_copy(em_h.at[pl.ds(base, _CK)], emvec)
                for g in range(_CK // 16):
                    o = g * 16
                    gidx[pl.ds(o, 16)] = sidx[pl.ds(o, 16)] + (v * _NP)
                    if v > 0:
                        wsc[pl.ds(o, 16)] = jnp.where(
                            emvec[pl.ds(o, 16)] > 0.3, wvec[pl.ds(o, 16)], 0.0)
                    else:
                        wsc[pl.ds(o, 16)] = wvec[pl.ds(o, 16)]
                pltpu.async_copy(table.at[gidx], rows, gsem).wait()

                def sbody(k, carry2):
                    kv = jnp.full((16,), k, jnp.int32)
                    wb = plsc.load_gather(wsc, [kv])
                    for j in range(_D // 16):
                        rows[k, pl.ds(j * 16, 16)] = (
                            rows[k, pl.ds(j * 16, 16)] * wb)
                    return carry2

                lax.fori_loop(0, _CK, sbody, 0)
                pltpu.sync_copy(rows, acc.at[didx], add=True)

            return carry

        lax.fori_loop(0, _TRIPS, chunk_body, 0)
        plsc.subcore_barrier()

        out_row = c * (3 * _NP) + v * _NP + row0
        pltpu.sync_copy(acc.at[pl.ds(row0, _STRIPE)],
                        out_h.at[pl.ds(out_row, _STRIPE)])


def _proj1_body(x_ref, w1s_ref, degp_ref, hs_ref, dinv_ref):
    degb = degp_ref[...]
    ssum = jnp.sum(degb, axis=0, keepdims=True) + 1.0
    dinv_row = lax.rsqrt(ssum)
    ri = lax.broadcasted_iota(jnp.int32, (_RB, _RB), 0)
    ci = lax.broadcasted_iota(jnp.int32, (_RB, _RB), 1)
    dmat = jnp.where(ri == ci, dinv_row, 0.0)
    h = jnp.dot(x_ref[...], w1s_ref[...],
                preferred_element_type=jnp.float32,
                precision=lax.Precision.HIGHEST)
    hs_ref[...] = jnp.dot(dmat, h,
                          preferred_element_type=jnp.float32,
                          precision=lax.Precision.HIGHEST)
    dinv_ref[...] = jnp.sum(dmat, axis=1, keepdims=True)


_proj1 = pl.pallas_call(
    _proj1_body,
    grid=(3, _NRB),
    in_specs=[
        pl.BlockSpec((_RB, _D), lambda v, r: (r, 0)),
        pl.BlockSpec((_D, _D), lambda v, r: (0, v)),
        pl.BlockSpec((_NW, _RB), lambda v, r: (0, v * _NRB + r)),
    ],
    out_specs=[
        pl.BlockSpec((_RB, _D), lambda v, r: (v * _NRB + r, 0)),
        pl.BlockSpec((_RB, 1), lambda v, r: (v * _NRB + r, 0)),
    ],
    out_shape=[
        jax.ShapeDtypeStruct((3 * _NP, _D), jnp.float32),
        jax.ShapeDtypeStruct((3 * _NP, 1), jnp.float32),
    ],
)


def _epi1_body(s0_ref, s1_ref, hs_ref, dinv_ref, b1_ref, w2_ref, gs_ref):
    dv = dinv_ref[...]
    a = (s0_ref[...] + s1_ref[...] + hs_ref[...]) * dv + b1_ref[...]
    a = jnp.maximum(a, 0.0)
    g = jnp.dot(a, w2_ref[...],
                preferred_element_type=jnp.float32,
                precision=lax.Precision.HIGHEST)
    gs_ref[...] = g * dv


_epi1 = pl.pallas_call(
    _epi1_body,
    grid=(3 * _NRB,),
    in_specs=[
        pl.BlockSpec((_RB, _D), lambda j: (j, 0)),
        pl.BlockSpec((_RB, _D), lambda j: (3 * _NRB + j, 0)),
        pl.BlockSpec((_RB, _D), lambda j: (j, 0)),
        pl.BlockSpec((_RB, 1), lambda j: (j, 0)),
        pl.BlockSpec((1, _D), lambda j: (0, 0)),
        pl.BlockSpec((_D, _D), lambda j: (0, 0)),
    ],
    out_specs=pl.BlockSpec((_RB, _D), lambda j: (j, 0)),
    out_shape=jax.ShapeDtypeStruct((3 * _NP, _D), jnp.float32),
)


def _epi2_body(s0_ref, s1_ref, gs_ref, dinv_ref, b2_ref, z_ref):
    dv = dinv_ref[...]
    a = (s0_ref[...] + s1_ref[...] + gs_ref[...]) * dv + b2_ref[...]
    z_ref[...] = jnp.maximum(a, 0.0)


_epi2 = pl.pallas_call(
    _epi2_body,
    grid=(3 * _NRB,),
    in_specs=[
        pl.BlockSpec((_RB, _D), lambda j: (j, 0)),
        pl.BlockSpec((_RB, _D), lambda j: (3 * _NRB + j, 0)),
        pl.BlockSpec((_RB, _D), lambda j: (j, 0)),
        pl.BlockSpec((_RB, 1), lambda j: (j, 0)),
        pl.BlockSpec((1, _D), lambda j: (0, 0)),
    ],
    out_specs=pl.BlockSpec((_RB, _D), lambda j: (j, 0)),
    out_shape=jax.ShapeDtypeStruct((3 * _NP, _D), jnp.float32),
)


def kernel(x, edge_index, edge_weight, feat_rand1, edge_rand1,
           feat_rand2, edge_rand2, W1, b1, W2, b2):
    src = edge_index[0]
    dst = edge_index[1]
    fm1 = (feat_rand1 > 0.3).astype(jnp.float32)
    fm2 = (feat_rand2 > 0.3).astype(jnp.float32)
    w1s = jnp.concatenate([W1, fm1[:, None] * W1, fm2[:, None] * W1], axis=1)

    degp = _deg_kernel(dst, edge_weight, edge_rand1, edge_rand2)
    degp = degp.reshape(_NW, 3 * _NP)

    xp = jnp.pad(x, ((0, _NP - _N), (0, 0)))
    hs, dinv = _proj1(xp, w1s, degp)
    s1 = _agg_kernel(hs, src, dst, edge_weight, edge_rand1, edge_rand2)
    gs = _epi1(s1, s1, hs, dinv, b1.reshape(1, _D), W2)
    s2 = _agg_kernel(gs, src, dst, edge_weight, edge_rand1, edge_rand2)
    z = _epi2(s2, s2, gs, dinv, b2.reshape(1, _D))
    return (z[:_N], z[_NP:_NP + _N], z[2 * _NP:2 * _NP + _N])


# speedup vs baseline: 10.9343x; 1.5457x over previous
import functools

import jax
import jax.numpy as jnp
from jax import lax
from jax.experimental import pallas as pl
from jax.experimental.pallas import tpu as pltpu
from jax.experimental.pallas import tpu_sc as plsc

_N = 10000
_NP = 10240
_E = 320000
_D = 128

_NC = 2
_NS = 16
_NW = _NC * _NS

_EPW = _E // _NW
_CK = 128
_NCH = _E // _CK
_TRIPS = (_NCH + _NW - 1) // _NW
_STRIPE = _NP // _NS

_RB = 512
_NRB = _NP // _RB

_mesh = plsc.VectorSubcoreMesh(core_axis_name="c", subcore_axis_name="s")


@functools.partial(
    pl.kernel,
    out_type=jax.ShapeDtypeStruct((_NW * 3 * _NP,), jnp.float32),
    mesh=_mesh,
    compiler_params=pltpu.CompilerParams(needs_layout_passes=False),
    scratch_types=[
        pltpu.VMEM((_EPW,), jnp.int32),
        pltpu.VMEM((_EPW,), jnp.float32),
        pltpu.VMEM((_EPW,), jnp.float32),
        pltpu.VMEM((_EPW,), jnp.float32),
        pltpu.VMEM((3 * _NP,), jnp.float32),
    ],
)
def _deg_kernel(dst_h, w_h, em1_h, em2_h, out_h, dstv, wv, e1v, e2v, acc):
    c = lax.axis_index("c")
    s = lax.axis_index("s")
    wid = s * _NC + c
    base = wid * _EPW

    zero = jnp.zeros((16,), jnp.float32)

    def zbody(m, carry):
        acc[pl.ds(m * 16, 16)] = zero
        return carry

    lax.fori_loop(0, (3 * _NP) // 16, zbody, 0)

    pltpu.sync_copy(dst_h.at[pl.ds(base, _EPW)], dstv)
    pltpu.sync_copy(w_h.at[pl.ds(base, _EPW)], wv)
    pltpu.sync_copy(em1_h.at[pl.ds(base, _EPW)], e1v)
    pltpu.sync_copy(em2_h.at[pl.ds(base, _EPW)], e2v)

    def ebody(g, carry):
        o = g * 16
        d16 = dstv[pl.ds(o, 16)]
        w16 = wv[pl.ds(o, 16)]
        e1 = e1v[pl.ds(o, 16)]
        e2 = e2v[pl.ds(o, 16)]
        plsc.addupdate_scatter(acc, [d16], w16)
        plsc.addupdate_scatter(acc, [d16 + _NP], w16, mask=e1 > 0.3)
        plsc.addupdate_scatter(acc, [d16 + 2 * _NP], w16, mask=e2 > 0.3)
        return carry

    lax.fori_loop(0, _EPW // 16, ebody, 0)

    pltpu.sync_copy(acc, out_h.at[pl.ds(wid * (3 * _NP), 3 * _NP)])


_HD = _D // 2
_CPS = _NCH // _NS
_LCH = 168
_LE = _LCH * _CK


@functools.partial(
    pl.kernel,
    out_type=jax.ShapeDtypeStruct((_NC * 3 * _NP, _HD), jnp.float32),
    mesh=_mesh,
    compiler_params=pltpu.CompilerParams(needs_layout_passes=False,
                                         use_tc_tiling_on_sc=False),
    scratch_types=[
        pltpu.VMEM((_LE,), jnp.int32),
        pltpu.VMEM((_LCH, _CK), jnp.int32),
        pltpu.VMEM((_LE,), jnp.float32),
        pltpu.VMEM((_CK,), jnp.int32),
        pltpu.VMEM((_CK,), jnp.int32),
        pltpu.VMEM((_CK,), jnp.float32),
        pltpu.VMEM((_CK,), jnp.float32),
        pltpu.VMEM((_CK,), jnp.float32),
        pltpu.VMEM((_CK, _HD), jnp.float32),
        pltpu.VMEM((_CK, _HD), jnp.float32),
        pltpu.VMEM_SHARED((_NP, _HD), jnp.float32),
        pltpu.SemaphoreType.DMA,
        pltpu.SemaphoreType.DMA,
    ],
)
def _agg_kernel(table, src_h, dst3_h, w_h, em1_h, em2_h, out_h,
                sloc, dloc, wloc, gidx0, gidx1, emb0, emb1, wsc,
                rows0, rows1, acc, gsem0, gsem1):
    c = lax.axis_index("c")
    s = lax.axis_index("s")
    row0 = s * _STRIPE

    n_s = jnp.where(s < 4, _CPS + 1, _CPS)
    s_s = _CPS * s + jnp.minimum(s, 4)
    abase = (s_s // 8) * 8
    doff = s_s - abase

    pltpu.sync_copy(src_h.at[pl.ds(abase * _CK, _LE)], sloc)
    pltpu.sync_copy(dst3_h.at[pl.ds(abase, _LCH)], dloc)
    pltpu.sync_copy(w_h.at[pl.ds(abase * _CK, _LE)], wloc)

    zero = jnp.zeros((16,), jnp.float32)
    gbufs = (gidx0, gidx1)
    ebufs = (emb0, emb1)
    bufs = (rows0, rows1)
    sems = (gsem0, gsem1)

    for v in range(3):
        em_h = (None, em1_h, em2_h)[v]
        goff = 2 * v * _NP + c

        def _prefetch(r, b):
            gb = gbufs[b]
            for g in range(_CK // 16):
                sl = pl.ds(g * 16, 16)
                s16 = sloc[pl.ds(r * _CK + g * 16, 16)]
                gb[sl] = s16 * 2 + goff
            pltpu.async_copy(table.at[gb], bufs[b], sems[b])
            if v > 0:
                pltpu.async_copy(em_h.at[pl.ds((abase + r) * _CK, _CK)],
                                 ebufs[b], sems[b])

        def _wait(r, b):
            pltpu.make_async_copy(table.at[gbufs[b]], bufs[b], sems[b]).wait()
            if v > 0:
                pltpu.make_async_copy(
                    em_h.at[pl.ds((abase + r) * _CK, _CK)],
                    ebufs[b], sems[b]).wait()

        def zb(i, carry):
            for j in range(_HD // 16):
                rows0[i, pl.ds(j * 16, 16)] = zero
            return carry

        lax.fori_loop(0, _CK, zb, 0)
        for j in range(_STRIPE // _CK):
            pltpu.sync_copy(rows0, acc.at[pl.ds(row0 + j * _CK, _CK)])
        plsc.subcore_barrier()

        _prefetch(doff, 0)
        _prefetch(doff + 1, 1)

        def pair(m, carry):
            i0 = m * 2
            for b in range(2):
                i = i0 + b
                buf = bufs[b]

                @pl.when(i < n_s)
                def _(i=i, b=b, buf=buf):
                    r = doff + i
                    _wait(r, b)
                    eb = ebufs[b]
                    for g in range(_CK // 16):
                        sl = pl.ds(g * 16, 16)
                        wv = wloc[pl.ds(r * _CK + g * 16, 16)]
                        if v > 0:
                            wsc[sl] = jnp.where(eb[sl] > 0.3, wv, 0.0)
                        else:
                            wsc[sl] = wv

                    def sbody(k, cr):
                        kv = jnp.full((16,), k, jnp.int32)
                        wb = plsc.load_gather(wsc, [kv])
                        for j in range(_HD // 16):
                            buf[k, pl.ds(j * 16, 16)] = (
                                buf[k, pl.ds(j * 16, 16)] * wb)
                        return cr

                    lax.fori_loop(0, _CK, sbody, 0)
                    pltpu.sync_copy(buf, acc.at[dloc.at[r]], add=True)

                    @pl.when(i + 2 < n_s)
                    def _(i=i, b=b):
                        _prefetch(doff + i + 2, b)

            return carry

        lax.fori_loop(0, (_CPS + 2) // 2, pair, 0)
        plsc.subcore_barrier()

        out_row = c * (3 * _NP) + v * _NP + row0
        pltpu.sync_copy(acc.at[pl.ds(row0, _STRIPE)],
                        out_h.at[pl.ds(out_row, _STRIPE)])


def _proj1_body(x_ref, w1s_ref, degp_ref, hs_ref, dinv_ref):
    degb = degp_ref[...]
    ssum = jnp.sum(degb, axis=0, keepdims=True) + 1.0
    dinv_row = lax.rsqrt(ssum)
    ri = lax.broadcasted_iota(jnp.int32, (_RB, _RB), 0)
    ci = lax.broadcasted_iota(jnp.int32, (_RB, _RB), 1)
    dmat = jnp.where(ri == ci, dinv_row, 0.0)
    h = jnp.dot(x_ref[...], w1s_ref[...],
                preferred_element_type=jnp.float32,
                precision=lax.Precision.HIGHEST)
    hs_ref[...] = jnp.dot(dmat, h,
                          preferred_element_type=jnp.float32,
                          precision=lax.Precision.HIGHEST)
    dinv_ref[...] = jnp.sum(dmat, axis=1, keepdims=True)


_proj1 = pl.pallas_call(
    _proj1_body,
    grid=(3, _NRB),
    in_specs=[
        pl.BlockSpec((_RB, _D), lambda v, r: (r, 0)),
        pl.BlockSpec((_D, _D), lambda v, r: (0, v)),
        pl.BlockSpec((_NW, _RB), lambda v, r: (0, v * _NRB + r)),
    ],
    out_specs=[
        pl.BlockSpec((_RB, _D), lambda v, r: (v * _NRB + r, 0)),
        pl.BlockSpec((_RB, 1), lambda v, r: (v * _NRB + r, 0)),
    ],
    out_shape=[
        jax.ShapeDtypeStruct((3 * _NP, _D), jnp.float32),
        jax.ShapeDtypeStruct((3 * _NP, 1), jnp.float32),
    ],
)


def _epi1_body(s0_ref, s1_ref, hs_ref, dinv_ref, b1_ref, w2_ref, gs_ref):
    dv = dinv_ref[...]
    sfull = jnp.concatenate([s0_ref[...], s1_ref[...]], axis=1)
    a = (sfull + hs_ref[...]) * dv + b1_ref[...]
    a = jnp.maximum(a, 0.0)
    g = jnp.dot(a, w2_ref[...],
                preferred_element_type=jnp.float32,
                precision=lax.Precision.HIGHEST)
    gs_ref[...] = g * dv


_epi1 = pl.pallas_call(
    _epi1_body,
    grid=(3 * _NRB,),
    in_specs=[
        pl.BlockSpec((_RB, _HD), lambda j: (j, 0)),
        pl.BlockSpec((_RB, _HD), lambda j: (3 * _NRB + j, 0)),
        pl.BlockSpec((_RB, _D), lambda j: (j, 0)),
        pl.BlockSpec((_RB, 1), lambda j: (j, 0)),
        pl.BlockSpec((1, _D), lambda j: (0, 0)),
        pl.BlockSpec((_D, _D), lambda j: (0, 0)),
    ],
    out_specs=pl.BlockSpec((_RB, _D), lambda j: (j, 0)),
    out_shape=jax.ShapeDtypeStruct((3 * _NP, _D), jnp.float32),
)


def _epi2_body(s0_ref, s1_ref, gs_ref, dinv_ref, b2_ref, z_ref):
    dv = dinv_ref[...]
    sfull = jnp.concatenate([s0_ref[...], s1_ref[...]], axis=1)
    a = (sfull + gs_ref[...]) * dv + b2_ref[...]
    z_ref[...] = jnp.maximum(a, 0.0)


_epi2 = pl.pallas_call(
    _epi2_body,
    grid=(3 * _NRB,),
    in_specs=[
        pl.BlockSpec((_RB, _HD), lambda j: (j, 0)),
        pl.BlockSpec((_RB, _HD), lambda j: (3 * _NRB + j, 0)),
        pl.BlockSpec((_RB, _D), lambda j: (j, 0)),
        pl.BlockSpec((_RB, 1), lambda j: (j, 0)),
        pl.BlockSpec((1, _D), lambda j: (0, 0)),
    ],
    out_specs=pl.BlockSpec((_RB, _D), lambda j: (j, 0)),
    out_shape=jax.ShapeDtypeStruct((3 * _NP, _D), jnp.float32),
)


def kernel(x, edge_index, edge_weight, feat_rand1, edge_rand1,
           feat_rand2, edge_rand2, W1, b1, W2, b2):
    src = edge_index[0]
    dst = edge_index[1]
    fm1 = (feat_rand1 > 0.3).astype(jnp.float32)
    fm2 = (feat_rand2 > 0.3).astype(jnp.float32)
    w1s = jnp.concatenate([W1, fm1[:, None] * W1, fm2[:, None] * W1], axis=1)

    degp = _deg_kernel(dst, edge_weight, edge_rand1, edge_rand2)
    degp = degp.reshape(_NW, 3 * _NP)

    xp = jnp.pad(x, ((0, _NP - _N), (0, 0)))
    hs, dinv = _proj1(xp, w1s, degp)
    epad = 2560 * _CK - _E
    dst3 = jnp.pad(dst.reshape(_NCH, _CK), ((0, 60), (0, 0)))
    src_p = jnp.pad(src, (0, epad))
    w_p = jnp.pad(edge_weight, (0, epad))
    em1_p = jnp.pad(edge_rand1, (0, epad))
    em2_p = jnp.pad(edge_rand2, (0, epad))
    s1 = _agg_kernel(hs.reshape(6 * _NP, _HD), src_p, dst3, w_p, em1_p,
                     em2_p)
    gs = _epi1(s1, s1, hs, dinv, b1.reshape(1, _D), W2)
    s2 = _agg_kernel(gs.reshape(6 * _NP, _HD), src_p, dst3, w_p, em1_p,
                      em2_p)
    z = _epi2(s2, s2, gs, dinv, b2.reshape(1, _D))
    return (z[:_N], z[_NP:_NP + _N], z[2 * _NP:2 * _NP + _N])


# speedup vs baseline: 12.8364x; 1.1740x over previous
import functools

import jax
import jax.numpy as jnp
from jax import lax
from jax.experimental import pallas as pl
from jax.experimental.pallas import tpu as pltpu
from jax.experimental.pallas import tpu_sc as plsc

_N = 10000
_NP = 10240
_E = 320000
_D = 128

_NC = 2
_NS = 16
_NW = _NC * _NS

_EPW = _E // _NW
_CK = 128
_NCH = _E // _CK
_TRIPS = (_NCH + _NW - 1) // _NW
_STRIPE = _NP // _NS

_RB = 512
_NRB = _NP // _RB

_mesh = plsc.VectorSubcoreMesh(core_axis_name="c", subcore_axis_name="s")


@functools.partial(
    pl.kernel,
    out_type=jax.ShapeDtypeStruct((_NW * 3 * _NP,), jnp.float32),
    mesh=_mesh,
    compiler_params=pltpu.CompilerParams(needs_layout_passes=False),
    scratch_types=[
        pltpu.VMEM((_EPW,), jnp.int32),
        pltpu.VMEM((_EPW,), jnp.float32),
        pltpu.VMEM((_EPW,), jnp.float32),
        pltpu.VMEM((_EPW,), jnp.float32),
        pltpu.VMEM((3 * _NP,), jnp.float32),
    ],
)
def _deg_kernel(dst_h, w_h, em1_h, em2_h, out_h, dstv, wv, e1v, e2v, acc):
    c = lax.axis_index("c")
    s = lax.axis_index("s")
    wid = s * _NC + c
    base = wid * _EPW

    zero = jnp.zeros((16,), jnp.float32)

    def zbody(m, carry):
        acc[pl.ds(m * 16, 16)] = zero
        return carry

    lax.fori_loop(0, (3 * _NP) // 16, zbody, 0)

    pltpu.sync_copy(dst_h.at[pl.ds(base, _EPW)], dstv)
    pltpu.sync_copy(w_h.at[pl.ds(base, _EPW)], wv)
    pltpu.sync_copy(em1_h.at[pl.ds(base, _EPW)], e1v)
    pltpu.sync_copy(em2_h.at[pl.ds(base, _EPW)], e2v)

    def ebody(g, carry):
        o = g * 16
        d16 = dstv[pl.ds(o, 16)]
        w16 = wv[pl.ds(o, 16)]
        e1 = e1v[pl.ds(o, 16)]
        e2 = e2v[pl.ds(o, 16)]
        plsc.addupdate_scatter(acc, [d16], w16)
        plsc.addupdate_scatter(acc, [d16 + _NP], w16, mask=e1 > 0.3)
        plsc.addupdate_scatter(acc, [d16 + 2 * _NP], w16, mask=e2 > 0.3)
        return carry

    lax.fori_loop(0, _EPW // 16, ebody, 0)

    pltpu.sync_copy(acc, out_h.at[pl.ds(wid * (3 * _NP), 3 * _NP)])


_HD = _D // 2
_CPS = _NCH // _NS
_LCH = 168
_LE = _LCH * _CK


@functools.partial(
    pl.kernel,
    out_type=jax.ShapeDtypeStruct((_NC * 3 * _NP, _HD), jnp.float32),
    mesh=_mesh,
    compiler_params=pltpu.CompilerParams(needs_layout_passes=False,
                                         use_tc_tiling_on_sc=False),
    scratch_types=[
        pltpu.VMEM((_LE,), jnp.int32),
        pltpu.VMEM((_LCH, _CK), jnp.int32),
        pltpu.VMEM((_CK,), jnp.int32),
        pltpu.VMEM((_CK,), jnp.int32),
        pltpu.VMEM((_CK,), jnp.int32),
        pltpu.VMEM((3 * _CK,), jnp.float32),
        pltpu.VMEM((3 * _CK,), jnp.float32),
        pltpu.VMEM((3 * _CK,), jnp.float32),
        pltpu.VMEM((_CK,), jnp.float32),
        pltpu.VMEM((_CK, _HD), jnp.float32),
        pltpu.VMEM((_CK, _HD), jnp.float32),
        pltpu.VMEM((_CK, _HD), jnp.float32),
        pltpu.VMEM_SHARED((_NP, _HD), jnp.float32),
        pltpu.SemaphoreType.DMA,
        pltpu.SemaphoreType.DMA,
        pltpu.SemaphoreType.DMA,
        pltpu.SemaphoreType.DMA,
        pltpu.SemaphoreType.DMA,
        pltpu.SemaphoreType.DMA,
    ],
)
def _agg_kernel(table, src_h, dst3_h, wem_h, out_h,
                sloc, dloc, gidx0, gidx1, gidx2, wem0, wem1, wem2, wsc,
                rows0, rows1, rows2, acc,
                gsem0, gsem1, gsem2, ssem0, ssem1, ssem2):
    c = lax.axis_index("c")
    s = lax.axis_index("s")
    row0 = s * _STRIPE

    n_s = jnp.where(s < 4, _CPS + 1, _CPS)
    s_s = _CPS * s + jnp.minimum(s, 4)
    abase = (s_s // 8) * 8
    doff = s_s - abase

    pltpu.sync_copy(src_h.at[pl.ds(abase * _CK, _LE)], sloc)
    pltpu.sync_copy(dst3_h.at[pl.ds(abase, _LCH)], dloc)

    zero = jnp.zeros((16,), jnp.float32)
    gbufs = (gidx0, gidx1, gidx2)
    wbufs = (wem0, wem1, wem2)
    bufs = (rows0, rows1, rows2)
    gsems = (gsem0, gsem1, gsem2)
    ssems = (ssem0, ssem1, ssem2)
    lanes = lax.iota(jnp.int32, 16)

    for v in range(3):
        goff = 2 * v * _NP + c

        def _prefetch(r, b):
            gb = gbufs[b]
            for g in range(_CK // 16):
                sl = pl.ds(g * 16, 16)
                s16 = sloc[pl.ds(r * _CK + g * 16, 16)]
                gb[sl] = s16 * 2 + goff
            pltpu.async_copy(table.at[gb], bufs[b], gsems[b])
            pltpu.async_copy(wem_h.at[pl.ds((abase + r) * 3 * _CK, 3 * _CK)],
                             wbufs[b], gsems[b])

        def _wait_gather(r, b):
            pltpu.make_async_copy(table.at[gbufs[b]], bufs[b],
                                  gsems[b]).wait()
            pltpu.make_async_copy(
                wem_h.at[pl.ds((abase + r) * 3 * _CK, 3 * _CK)],
                wbufs[b], gsems[b]).wait()

        def _wait_scatter(b):
            pltpu.make_async_copy(bufs[b], acc.at[dloc.at[doff]],
                                  ssems[b]).wait()

        def zb(i, carry):
            for j in range(_HD // 16):
                rows0[i, pl.ds(j * 16, 16)] = zero
            return carry

        lax.fori_loop(0, _CK, zb, 0)
        for j in range(_STRIPE // _CK):
            pltpu.sync_copy(rows0, acc.at[pl.ds(row0 + j * _CK, _CK)])
        plsc.subcore_barrier()

        _prefetch(doff, 0)
        _prefetch(doff + 1, 1)

        def triple(m, carry):
            i0 = m * 3
            for t in range(3):
                i = i0 + t
                buf = bufs[t]
                wb_ = wbufs[t]

                @pl.when(i < n_s)
                def _(i=i, t=t, buf=buf, wb_=wb_):
                    r = doff + i
                    _wait_gather(r, t)
                    for g in range(_CK // 16):
                        sl = pl.ds(g * 16, 16)
                        idx3 = lanes * 3 + (g * 48)
                        w16 = plsc.load_gather(wb_, [idx3])
                        if v > 0:
                            em16 = plsc.load_gather(wb_, [idx3 + v])
                            wsc[sl] = jnp.where(em16 > 0.3, w16, 0.0)
                        else:
                            wsc[sl] = w16

                    def sbody(kk, cr):
                        k0 = kk * 8
                        for dk in range(8):
                            k = k0 + dk
                            kv = jnp.full((16,), k, jnp.int32)
                            wbv = plsc.load_gather(wsc, [kv])
                            for j in range(_HD // 16):
                                buf[k, pl.ds(j * 16, 16)] = (
                                    buf[k, pl.ds(j * 16, 16)] * wbv)
                        return cr

                    lax.fori_loop(0, _CK // 8, sbody, 0)
                    pltpu.async_copy(buf, acc.at[dloc.at[r]], ssems[t],
                                     add=True)

                    t2 = (t + 2) % 3

                    @pl.when(i + 2 < n_s)
                    def _(i=i, t2=t2):
                        @pl.when(i >= 1)
                        def _():
                            _wait_scatter(t2)

                        _prefetch(doff + i + 2, t2)

            return carry

        lax.fori_loop(0, (_CPS + 3) // 3, triple, 0)
        for t in range(3):
            _wait_scatter(t)
        plsc.subcore_barrier()

        out_row = c * (3 * _NP) + v * _NP + row0
        pltpu.sync_copy(acc.at[pl.ds(row0, _STRIPE)],
                        out_h.at[pl.ds(out_row, _STRIPE)])


def _proj1_body(x_ref, w1s_ref, degp_ref, hs_ref, dinv_ref):
    degb = degp_ref[...]
    ssum = jnp.sum(degb, axis=0, keepdims=True) + 1.0
    dinv_row = lax.rsqrt(ssum)
    ri = lax.broadcasted_iota(jnp.int32, (_RB, _RB), 0)
    ci = lax.broadcasted_iota(jnp.int32, (_RB, _RB), 1)
    dmat = jnp.where(ri == ci, dinv_row, 0.0)
    h = jnp.dot(x_ref[...], w1s_ref[...],
                preferred_element_type=jnp.float32,
                precision=lax.Precision.HIGHEST)
    dcol = jnp.sum(dmat, axis=1, keepdims=True)
    hs_ref[...] = h * dcol
    dinv_ref[...] = dcol


_proj1 = pl.pallas_call(
    _proj1_body,
    grid=(3, _NRB),
    in_specs=[
        pl.BlockSpec((_RB, _D), lambda v, r: (r, 0)),
        pl.BlockSpec((_D, _D), lambda v, r: (0, v)),
        pl.BlockSpec((_NW, _RB), lambda v, r: (0, v * _NRB + r)),
    ],
    out_specs=[
        pl.BlockSpec((_RB, _D), lambda v, r: (v * _NRB + r, 0)),
        pl.BlockSpec((_RB, 1), lambda v, r: (v * _NRB + r, 0)),
    ],
    out_shape=[
        jax.ShapeDtypeStruct((3 * _NP, _D), jnp.float32),
        jax.ShapeDtypeStruct((3 * _NP, 1), jnp.float32),
    ],
)


def _epi1_body(s0_ref, s1_ref, hs_ref, dinv_ref, b1_ref, w2_ref, gs_ref):
    dv = dinv_ref[...]
    sfull = jnp.concatenate([s0_ref[...], s1_ref[...]], axis=1)
    a = (sfull + hs_ref[...]) * dv + b1_ref[...]
    a = jnp.maximum(a, 0.0)
    g = jnp.dot(a, w2_ref[...],
                preferred_element_type=jnp.float32,
                precision=lax.Precision.HIGHEST)
    gs_ref[...] = g * dv


_epi1 = pl.pallas_call(
    _epi1_body,
    grid=(3 * _NRB,),
    in_specs=[
        pl.BlockSpec((_RB, _HD), lambda j: (j, 0)),
        pl.BlockSpec((_RB, _HD), lambda j: (3 * _NRB + j, 0)),
        pl.BlockSpec((_RB, _D), lambda j: (j, 0)),
        pl.BlockSpec((_RB, 1), lambda j: (j, 0)),
        pl.BlockSpec((1, _D), lambda j: (0, 0)),
        pl.BlockSpec((_D, _D), lambda j: (0, 0)),
    ],
    out_specs=pl.BlockSpec((_RB, _D), lambda j: (j, 0)),
    out_shape=jax.ShapeDtypeStruct((3 * _NP, _D), jnp.float32),
)


def _epi2_body(s0_ref, s1_ref, gs_ref, dinv_ref, b2_ref, z_ref):
    dv = dinv_ref[...]
    sfull = jnp.concatenate([s0_ref[...], s1_ref[...]], axis=1)
    a = (sfull + gs_ref[...]) * dv + b2_ref[...]
    z_ref[...] = jnp.maximum(a, 0.0)


_epi2 = pl.pallas_call(
    _epi2_body,
    grid=(3 * _NRB,),
    in_specs=[
        pl.BlockSpec((_RB, _HD), lambda j: (j, 0)),
        pl.BlockSpec((_RB, _HD), lambda j: (3 * _NRB + j, 0)),
        pl.BlockSpec((_RB, _D), lambda j: (j, 0)),
        pl.BlockSpec((_RB, 1), lambda j: (j, 0)),
        pl.BlockSpec((1, _D), lambda j: (0, 0)),
    ],
    out_specs=pl.BlockSpec((_RB, _D), lambda j: (j, 0)),
    out_shape=jax.ShapeDtypeStruct((3 * _NP, _D), jnp.float32),
)


def kernel(x, edge_index, edge_weight, feat_rand1, edge_rand1,
           feat_rand2, edge_rand2, W1, b1, W2, b2):
    src = edge_index[0]
    dst = edge_index[1]
    fm1 = (feat_rand1 > 0.3).astype(jnp.float32)
    fm2 = (feat_rand2 > 0.3).astype(jnp.float32)
    w1s = jnp.concatenate([W1, fm1[:, None] * W1, fm2[:, None] * W1], axis=1)

    degp = _deg_kernel(dst, edge_weight, edge_rand1, edge_rand2)
    degp = degp.reshape(_NW, 3 * _NP)

    xp = jnp.pad(x, ((0, _NP - _N), (0, 0)))
    hs, dinv = _proj1(xp, w1s, degp)
    epad = 2560 * _CK - _E
    dst3 = jnp.pad(dst.reshape(_NCH, _CK), ((0, 60), (0, 0)))
    src_p = jnp.pad(src, (0, epad))
    wem = jnp.pad(
        jnp.stack([edge_weight, edge_rand1, edge_rand2], axis=1).reshape(-1),
        (0, 3 * epad))
    s1 = _agg_kernel(hs.reshape(6 * _NP, _HD), src_p, dst3, wem)
    gs = _epi1(s1, s1, hs, dinv, b1.reshape(1, _D), W2)
    s2 = _agg_kernel(gs.reshape(6 * _NP, _HD), src_p, dst3, wem)
    z = _epi2(s2, s2, gs, dinv, b2.reshape(1, _D))
    return (z[:_N], z[_NP:_NP + _N], z[2 * _NP:2 * _NP + _N])
